# baseline (speedup 1.0000x reference)
import functools

import jax
import jax.numpy as jnp
from jax import lax
from jax.experimental import pallas as pl
from jax.experimental.pallas import tpu as pltpu
from jax.experimental.pallas import tpu_sc as plsc

_C = 384
_NS = 1024
_WS = 4
_WNH = 8
_NW = 64
_K = 16
_G = 24
_GD = 4 * _G
_NT = _K * _WS * _WS
_NB = 8

_RHS_T = (((1,), (1,)), ((), ()))
_MM = (((1,), (0,)), ((), ()))


def _select_body(z_ref, x_ref, idx_ref):
    ti = lax.broadcasted_iota(jnp.int32, (_NW, _NS), 1)
    wi = lax.broadcasted_iota(jnp.int32, (_NW, _NS), 0)
    q_sel = ((ti // 128 == wi // _WNH)
             & ((ti % 32) // _WS == wi % _WNH)).astype(jnp.float32)
    eye = (lax.broadcasted_iota(jnp.int32, (_NW, _NW), 0)
           == lax.broadcasted_iota(jnp.int32, (_NW, _NW), 1)).astype(
        jnp.float32)
    wi64 = lax.broadcasted_iota(jnp.int32, (_NW, _NW), 0)
    ji64 = lax.broadcasted_iota(jnp.int32, (_NW, _NW), 1)
    fidx_row = lax.broadcasted_iota(jnp.int32, (1, _NW), 1).astype(jnp.float32)
    qlane = lax.broadcasted_iota(jnp.int32, (_NW, _NT), 1)
    q_row = lax.broadcasted_iota(jnp.int32, (1, _NT), 1)
    pid = pl.program_id(0)

    for i in range(_NB):
        z = z_ref[i]
        zmax = jnp.max(z, axis=0, keepdims=True)
        sim = lax.dot_general(zmax, x_ref[i], _RHS_T,
                              preferred_element_type=jnp.float32)
        win_row = lax.dot_general(sim, q_sel, _RHS_T,
                                  preferred_element_type=jnp.float32,
                                  precision=lax.Precision.HIGHEST)
        win_col = lax.dot_general(eye, win_row, _RHS_T,
                                  preferred_element_type=jnp.float32,
                                  precision=lax.Precision.HIGHEST)
        vj = jnp.broadcast_to(win_row, (_NW, _NW))
        vw = jnp.broadcast_to(win_col, (_NW, _NW))
        beats = (vj > vw) | ((vj == vw) & (ji64 < wi64))
        rank = jnp.sum(beats.astype(jnp.int32), axis=1, keepdims=True)
        sel_t = (jnp.broadcast_to(rank, (_NW, _NT))
                 == qlane // _K).astype(jnp.float32)
        w_q = lax.dot_general(fidx_row, sel_t, _MM,
                              preferred_element_type=jnp.float32)
        w_qi = w_q.astype(jnp.int32)
        r_q = (q_row // _WS) % _WS
        c_q = q_row % _WS
        gb = pid * _NB + i
        idx_ref[pl.ds(i, 1), :] = (gb * _NS + (w_qi // _WNH) * 128
                                   + (w_qi % _WNH) * _WS + r_q * 32 + c_q)


def _select(z, x):
    B = z.shape[0]
    return pl.pallas_call(
        _select_body,
        grid=(B // _NB,),
        in_specs=[
            pl.BlockSpec((_NB, z.shape[1], _C), lambda b: (b, 0, 0)),
            pl.BlockSpec((_NB, _NS, _C), lambda b: (b, 0, 0)),
        ],
        out_specs=pl.BlockSpec((_NB, _NT), lambda b: (b, 0)),
        out_shape=jax.ShapeDtypeStruct((B, _NT), jnp.int32),
    )(z, x)


_SC_CHUNK = 128


def _sc_gather(table, idx_flat):
    n = idx_flat.shape[0]
    info = plsc.get_sparse_core_info()
    nw = info.num_cores * info.num_subcores
    per_w = n // nw
    nchunk = per_w // _SC_CHUNK
    mesh = plsc.VectorSubcoreMesh(core_axis_name="c", subcore_axis_name="s")

    @functools.partial(
        pl.kernel,
        out_type=jax.ShapeDtypeStruct((n, _C), jnp.float32),
        mesh=mesh,
        scratch_types=[
            pltpu.VMEM((_SC_CHUNK,), jnp.int32),
            pltpu.VMEM((_SC_CHUNK, _C), jnp.float32),
            pltpu.SemaphoreType.DMA,
        ],
    )
    def gather_kernel(table_hbm, idx_hbm, out_hbm, idx_v, rows_v, sem):
        wid = lax.axis_index("s") * info.num_cores + lax.axis_index("c")
        for chunk in range(nchunk):
            base = wid * per_w + chunk * _SC_CHUNK
            pltpu.sync_copy(idx_hbm.at[pl.ds(base, _SC_CHUNK)], idx_v)
            pltpu.async_copy(table_hbm.at[idx_v], rows_v, sem).wait()
            pltpu.sync_copy(rows_v, out_hbm.at[pl.ds(base, _SC_CHUNK)])

    return gather_kernel(table, idx_flat)


def _dense_body(xe_ref, wd_ref, bd_ref, wu_ref, bu_ref, out_ref):
    rows = _NB * _NT
    xe = xe_ref[...]
    wd = wd_ref[...]
    bd = bd_ref[...]
    wu = wu_ref[...]
    bu = bu_ref[...]
    t = lax.dot_general(xe, wd, _RHS_T,
                        preferred_element_type=jnp.float32) + bd
    qi = lax.broadcasted_iota(jnp.int32, (rows, _GD), 0)
    li = lax.broadcasted_iota(jnp.int32, (rows, _GD), 1)
    c_tok = qi % _WS
    r_tok = (qi // _WS) % _WS
    z1 = jnp.zeros((1, _GD), jnp.float32)
    z4 = jnp.zeros((_WS, _GD), jnp.float32)
    tm1 = jnp.concatenate([t[1:], z1], axis=0)
    tp1 = jnp.concatenate([z1, t[:-1]], axis=0)
    tm4 = jnp.concatenate([t[_WS:], z4], axis=0)
    tp4 = jnp.concatenate([z4, t[:-_WS]], axis=0)
    g0 = jnp.where(c_tok < _WS - 1, tm1, 0.0)
    g1 = jnp.where(c_tok > 0, tp1, 0.0)
    g2 = jnp.where(r_tok < _WS - 1, tm4, 0.0)
    g3 = jnp.where(r_tok > 0, tp4, 0.0)
    s = jnp.where(li < _G, g0,
                  jnp.where(li < 2 * _G, g1,
                            jnp.where(li < 3 * _G, g2, g3)))
    su = lax.dot_general(s, wu, _RHS_T,
                         preferred_element_type=jnp.float32)
    out_ref[...] = xe + su + bu


def _dense(xe_flat, w_down, bd, w_up, bu):
    n = xe_flat.shape[0]
    rows = _NB * _NT
    return pl.pallas_call(
        _dense_body,
        grid=(n // rows,),
        in_specs=[
            pl.BlockSpec((rows, _C), lambda b: (b, 0)),
            pl.BlockSpec((_GD, _C), lambda b: (0, 0)),
            pl.BlockSpec((1, _GD), lambda b: (0, 0)),
            pl.BlockSpec((_C, _GD), lambda b: (0, 0)),
            pl.BlockSpec((1, _C), lambda b: (0, 0)),
        ],
        out_specs=pl.BlockSpec((rows, _C), lambda b: (b, 0)),
        out_shape=jax.ShapeDtypeStruct((n, _C), jnp.float32),
    )(xe_flat, w_down, bd, w_up, bu)


def kernel(z, x, w_down, b_down, w_up, b_up):
    B = z.shape[0]
    bd = b_down.reshape(1, _GD)
    bu = b_up.reshape(1, _C)
    idx = _select(z, x)
    xe_flat = _sc_gather(x.reshape(B * _NS, _C), idx.reshape(B * _NT))
    out = _dense(xe_flat, w_down, bd, w_up, bu)
    return out.reshape(B, _NT, _C)
